# Initial kernel scaffold; baseline (speedup 1.0000x reference)
import jax
import jax.numpy as jnp
from jax.experimental import pallas as pl


def kernel(feat, edge_index, Wq, bq, Wk, bk, Wv, bv, ln_g, ln_b, W1, bf1, alpha, W2, bf2):
    raise NotImplementedError("write your pallas kernel here")


# speedup vs baseline: 13.3135x; 13.3135x over previous
import jax
import jax.numpy as jnp
from jax import lax
from jax.experimental import pallas as pl
from jax.experimental.pallas import tpu as pltpu
from jax.experimental.pallas import tpu_sc as plsc

_N = 10000
_E = 320000
_D = 128
_H = 8
_DH = 16
_DFF = 512

_NPAD = 10240
_NW = 32
_EPW = _E // _NW
_C = 40
_NCH = _EPW // _C
_CS = 48
_RPT = _NPAD // 16
_DR = _NPAD // 16
_DRPT = _DR // 16


def _proj_body(feat_ref, wq_ref, bq_ref, wk_ref, bk_ref, wv_ref, bv_ref,
               dst_ref, q_ref, k_ref, v_ref, d16_ref, a0_ref, r8_ref):
    x = feat_ref[...]
    dv = dst_ref[...]
    d16_ref[...] = lax.shift_right_logical(dv, 4)
    a0_ref[...] = lax.shift_left(lax.shift_right_logical(dv & 15, 1), 4)
    r8_ref[...] = lax.shift_left(dv & 1, 3)
    scale = jnp.float32(1.0) / jnp.sqrt(jnp.float32(_D))
    q_ref[...] = (jnp.dot(x, wq_ref[...], preferred_element_type=jnp.float32)
                  + bq_ref[...]) * scale
    k_ref[...] = (jnp.dot(x, wk_ref[...], preferred_element_type=jnp.float32)
                  + bk_ref[...])
    v_ref[...] = (jnp.dot(x, wv_ref[...], preferred_element_type=jnp.float32)
                  + bv_ref[...])


def _proj(feat, Wq, bq, Wk, bk, Wv, bv, dst3):
    grid = (_N // 1000,)
    row_spec = pl.BlockSpec((1000, _D), lambda i: (i, 0))
    w_spec = pl.BlockSpec((_D, _D), lambda i: (0, 0))
    b_spec = pl.BlockSpec((1, _D), lambda i: (0, 0))
    e_spec = pl.BlockSpec((1, 1, _E // 10), lambda i: (i, 0, 0))
    return pl.pallas_call(
        _proj_body,
        grid=grid,
        in_specs=[row_spec, w_spec, b_spec, w_spec, b_spec, w_spec, b_spec,
                  e_spec],
        out_specs=[row_spec, row_spec, row_spec, e_spec, e_spec, e_spec],
        out_shape=[jax.ShapeDtypeStruct((_N, _D), jnp.float32)] * 3
        + [jax.ShapeDtypeStruct((10, 1, _E // 10), jnp.int32)] * 3,
    )(feat, Wq, bq, Wk, bk, Wv, bv, dst3)


def _edge_body(q_hbm, k_hbm, v_hbm, src_hbm, dst_hbm, d16_hbm, a0_hbm,
               r8_hbm, out_v_hbm, out_d_hbm,
               src_v, dst_v, d16_v, a0_v, r8_v, kb, qb, vb, pb, ee_t, wvb, db,
               acc_v, acc_d, sem):
    c = lax.axis_index("c")
    s = lax.axis_index("s")
    wid = c * 16 + s
    zvec = jnp.zeros((16,), jnp.float32)
    iota16 = lax.broadcasted_iota(jnp.int32, (16,), 0)

    def _zrow(i, carry):
        for j in range(_D // 16):
            wvb[i, pl.ds(j * 16, 16)] = zvec
        return carry
    lax.fori_loop(0, _C, _zrow, 0)

    for j in range(_CS // 16):
        ee_t[pl.ds(_H * _CS + j * 16, 16)] = zvec

    base_r = s * _RPT
    for t in range(_RPT // _C):
        pltpu.sync_copy(wvb, acc_v.at[pl.ds(base_r + t * _C, _C)])
    pltpu.sync_copy(wvb, acc_d.at[pl.ds(s * _DRPT, _DRPT)])
    plsc.subcore_barrier()

    ew_base = wid * _EPW

    def _chunk(ci, carry):
        base = ew_base + ci * _C
        pltpu.sync_copy(src_hbm.at[pl.ds(base, _C)], src_v)
        pltpu.sync_copy(dst_hbm.at[pl.ds(base, _C)], dst_v)
        pltpu.sync_copy(d16_hbm.at[pl.ds(base, _C)], d16_v)
        pltpu.sync_copy(a0_hbm.at[pl.ds(base, _C)], a0_v)
        pltpu.sync_copy(r8_hbm.at[pl.ds(base, _C)], r8_v)
        pltpu.async_copy(k_hbm.at[src_v], kb, sem).wait()
        pltpu.async_copy(q_hbm.at[dst_v], qb, sem).wait()
        pltpu.async_copy(v_hbm.at[src_v], vb, sem).wait()

        pltpu.async_copy(k_hbm.at[src_v], kb, sem).wait()
        pltpu.async_copy(q_hbm.at[dst_v], qb, sem).wait()
        pltpu.async_copy(v_hbm.at[src_v], vb, sem).wait()

        def _didx(g, carry):
            dv = dst_v[pl.ds(g * 16, 16)]
            a0_v[pl.ds(g * 16, 16)] = lax.shift_left(
                lax.shift_right_logical(dv & 15, 1), 4)
            r8_v[pl.ds(g * 16, 16)] = lax.shift_left(dv & 1, 3)
            return carry
        lax.fori_loop(0, _C // 16, _didx, 0)


        def _prod(e, carry):
            for h in range(_H):
                prod = kb[e, pl.ds(h * 16, 16)] * qb[e, pl.ds(h * 16, 16)]
                idx = (h * 16 + iota16) * _CS + e
                plsc.store_scatter(pb, [idx], prod)
            return carry
        lax.fori_loop(0, _C, _prod, 0)

        def _dot(t, carry):
            g = t // _H
            h = t % _H
            acc = jnp.zeros((16,), jnp.float32)
            for d in range(16):
                acc = acc + pb[pl.ds((h * 16 + d) * _CS + g * 16, 16)]
            ee = jnp.exp(acc)
            ee_t[pl.ds(h * _CS + g * 16, 16)] = ee
            return carry
        lax.fori_loop(0, (_CS // 16) * _H, _dot, 0)

        def _medge(e, carry):
            a0 = plsc.load_gather(a0_v, [jnp.zeros((16,), jnp.int32) + e])
            r8 = plsc.load_gather(r8_v, [jnp.zeros((16,), jnp.int32) + e])
            valid = (iota16 >= r8) & (iota16 < r8 + 8)
            hidx = jnp.where(valid, iota16 - r8, _H)
            w = plsc.load_gather(ee_t, [hidx * _CS + e])
            for j in range(_D // 16):
                db[e, pl.ds(j * 16, 16)] = jnp.where(a0 == j * 16, w, zvec)
            for h in range(_H):
                eev = plsc.load_gather(
                    ee_t, [jnp.full((16,), h * _CS, jnp.int32) + e])
                wvb[e, pl.ds(h * 16, 16)] = vb[e, pl.ds(h * 16, 16)] * eev
            return carry
        lax.fori_loop(0, _C, _medge, 0)

        pltpu.sync_copy(wvb, acc_v.at[dst_v], add=True)
        pltpu.sync_copy(db, acc_d.at[d16_v], add=True)
        return carry
    lax.fori_loop(0, _NCH, _chunk, 0)

    plsc.subcore_barrier()
    for t in range(_RPT // _C):
        r0 = base_r + t * _C
        pltpu.sync_copy(acc_v.at[pl.ds(r0, _C)], wvb)
        pltpu.sync_copy(wvb, out_v_hbm.at[c, pl.ds(r0, _C)])
    d0 = s * _DRPT
    pltpu.sync_copy(acc_d.at[pl.ds(d0, _DRPT)], wvb)
    pltpu.sync_copy(wvb, out_d_hbm.at[c, pl.ds(d0, _DRPT)])


def _edge(q, k, v, src, dst, d16, a0, r8):
    mesh = plsc.VectorSubcoreMesh(core_axis_name="c", subcore_axis_name="s")
    fn = pl.kernel(
        _edge_body,
        out_type=[
            jax.ShapeDtypeStruct((2, _NPAD, _D), jnp.float32),
            jax.ShapeDtypeStruct((2, _DR, _D), jnp.float32),
        ],
        mesh=mesh,
        compiler_params=pltpu.CompilerParams(needs_layout_passes=False),
        scratch_types=[
            pltpu.VMEM((_C,), jnp.int32),
            pltpu.VMEM((_C,), jnp.int32),
            pltpu.VMEM((_C,), jnp.int32),
            pltpu.VMEM((_C,), jnp.int32),
            pltpu.VMEM((_C,), jnp.int32),
            pltpu.VMEM((_C, _D), jnp.float32),
            pltpu.VMEM((_C, _D), jnp.float32),
            pltpu.VMEM((_C, _D), jnp.float32),
            pltpu.VMEM((_CS * _D,), jnp.float32),
            pltpu.VMEM(((_H + 1) * _CS,), jnp.float32),
            pltpu.VMEM((_C, _D), jnp.float32),
            pltpu.VMEM((_C, _D), jnp.float32),
            pltpu.VMEM_SHARED((_NPAD, _D), jnp.float32),
            pltpu.VMEM_SHARED((_DR, _D), jnp.float32),
            pltpu.SemaphoreType.DMA,
        ],
    )
    return fn(q, k, v, src, dst, d16, a0, r8)


def _ln(x, g, b):
    mu = jnp.mean(x, axis=1, keepdims=True)
    var = jnp.mean((x - mu) * (x - mu), axis=1, keepdims=True)
    return (x - mu) * lax.rsqrt(var + 1e-5) * g + b


def _comb_body(pv_ref, pd_ref, feat_ref, lng_ref, lnb_ref, w1_ref, b1_ref,
               al_ref, w2_ref, b2_ref, o_ref):
    pv = pv_ref[...]
    sacc = pv[0] + pv[1]
    pd = pd_ref[...]
    den = pd[0] + pd[1]
    den = jnp.where(den == 0.0, jnp.float32(1.0), den)
    hh = lax.broadcasted_iota(jnp.int32, (_H, _D), 0)
    jj = lax.broadcasted_iota(jnp.int32, (_H, _D), 1)
    expand = (jj // _DH == hh).astype(jnp.float32)
    den16 = jnp.dot(den, expand, preferred_element_type=jnp.float32)
    rst = sacc / den16 + feat_ref[...]
    g = lng_ref[...]
    b = lnb_ref[...]
    rst = _ln(rst, g, b)
    h1 = jnp.dot(rst, w1_ref[...], preferred_element_type=jnp.float32) + b1_ref[...]
    h1 = jnp.where(h1 >= 0, h1, al_ref[...] * h1)
    ffn = jnp.dot(h1, w2_ref[...], preferred_element_type=jnp.float32) + b2_ref[...]
    o_ref[...] = _ln(rst + ffn, g, b)


def _combine(pv, pd, feat, ln_g, ln_b, W1, bf1, alpha, W2, bf2):
    grid = (_N // 1000,)
    return pl.pallas_call(
        _comb_body,
        grid=grid,
        in_specs=[
            pl.BlockSpec((2, 1000, _D), lambda i: (0, i, 0)),
            pl.BlockSpec((2, 1000, _H), lambda i: (0, i, 0)),
            pl.BlockSpec((1000, _D), lambda i: (i, 0)),
            pl.BlockSpec((1, _D), lambda i: (0, 0)),
            pl.BlockSpec((1, _D), lambda i: (0, 0)),
            pl.BlockSpec((_D, _DFF), lambda i: (0, 0)),
            pl.BlockSpec((1, _DFF), lambda i: (0, 0)),
            pl.BlockSpec((1, _DFF), lambda i: (0, 0)),
            pl.BlockSpec((_DFF, _D), lambda i: (0, 0)),
            pl.BlockSpec((1, _D), lambda i: (0, 0)),
        ],
        out_specs=pl.BlockSpec((1000, _D), lambda i: (i, 0)),
        out_shape=jax.ShapeDtypeStruct((_N, _D), jnp.float32),
    )(pv, pd, feat, ln_g, ln_b, W1, bf1, alpha, W2, bf2)


def kernel(feat, edge_index, Wq, bq, Wk, bk, Wv, bv, ln_g, ln_b, W1, bf1,
           alpha, W2, bf2):
    ei = edge_index.astype(jnp.int32)
    dst3 = ei[1].reshape(10, 1, _E // 10)
    q, k, v, d16_3d, a0_3d, r8_3d = _proj(
        feat, Wq, bq.reshape(1, _D), Wk, bk.reshape(1, _D),
        Wv, bv.reshape(1, _D), dst3)
    src = ei[0]
    dst = ei[1]
    pv, pd_raw = _edge(q, k, v, src, dst, d16_3d.reshape(_E),
                       a0_3d.reshape(_E), r8_3d.reshape(_E))
    pd = pd_raw.reshape(2, _NPAD, _H)
    return _combine(pv, pd, feat, ln_g.reshape(1, _D), ln_b.reshape(1, _D),
                    W1, bf1.reshape(1, _DFF), alpha.reshape(1, _DFF),
                    W2, bf2.reshape(1, _D))


# speedup vs baseline: 16.6656x; 1.2518x over previous
import jax
import jax.numpy as jnp
from jax import lax
from jax.experimental import pallas as pl
from jax.experimental.pallas import tpu as pltpu
from jax.experimental.pallas import tpu_sc as plsc

_N = 10000
_E = 320000
_D = 128
_H = 8
_DH = 16
_DFF = 512

_NPAD = 10240
_NW = 32
_EPW = _E // _NW
_C = 40
_NCH = _EPW // _C
_CS = 48
_RPT = _NPAD // 16
_DR = _NPAD // 16
_DRPT = _DR // 16


def _proj_body(feat_ref, wq_ref, bq_ref, wk_ref, bk_ref, wv_ref, bv_ref,
               dst_ref, q_ref, k_ref, v_ref, d16_ref, a0_ref, r8_ref):
    x = feat_ref[...]
    dv = dst_ref[...]
    d16_ref[...] = lax.shift_right_logical(dv, 4)
    a0_ref[...] = lax.shift_left(lax.shift_right_logical(dv & 15, 1), 4)
    r8_ref[...] = lax.shift_left(dv & 1, 3)
    scale = jnp.float32(1.0) / jnp.sqrt(jnp.float32(_D))
    q_ref[...] = (jnp.dot(x, wq_ref[...], preferred_element_type=jnp.float32)
                  + bq_ref[...]) * scale
    k_ref[...] = (jnp.dot(x, wk_ref[...], preferred_element_type=jnp.float32)
                  + bk_ref[...])
    v_ref[...] = (jnp.dot(x, wv_ref[...], preferred_element_type=jnp.float32)
                  + bv_ref[...])


def _proj(feat, Wq, bq, Wk, bk, Wv, bv, dst3):
    grid = (_N // 1000,)
    row_spec = pl.BlockSpec((1000, _D), lambda i: (i, 0))
    w_spec = pl.BlockSpec((_D, _D), lambda i: (0, 0))
    b_spec = pl.BlockSpec((1, _D), lambda i: (0, 0))
    e_spec = pl.BlockSpec((1, 1, _E // 10), lambda i: (i, 0, 0))
    return pl.pallas_call(
        _proj_body,
        grid=grid,
        in_specs=[row_spec, w_spec, b_spec, w_spec, b_spec, w_spec, b_spec,
                  e_spec],
        out_specs=[row_spec, row_spec, row_spec, e_spec, e_spec, e_spec],
        out_shape=[jax.ShapeDtypeStruct((_N, _D), jnp.float32)] * 3
        + [jax.ShapeDtypeStruct((10, 1, _E // 10), jnp.int32)] * 3,
    )(feat, Wq, bq, Wk, bk, Wv, bv, dst3)


def _edge_body(q_hbm, k_hbm, v_hbm, src_hbm, dst_hbm, d16_hbm, a0_hbm,
               r8_hbm, out_v_hbm, out_d_hbm,
               src_v, dst_v, d16_v, a0_v, r8_v, kb, qb, vb, pb, ee_t, wvb, db,
               acc_v, acc_d, sem, sem2, sem3):
    c = lax.axis_index("c")
    s = lax.axis_index("s")
    wid = c * 16 + s
    zvec = jnp.zeros((16,), jnp.float32)
    iota16 = lax.broadcasted_iota(jnp.int32, (16,), 0)

    def _zrow(i, carry):
        for j in range(_D // 16):
            wvb[i, pl.ds(j * 16, 16)] = zvec
        return carry
    lax.fori_loop(0, _C, _zrow, 0)

    for j in range(_CS // 16):
        ee_t[pl.ds(_H * _CS + j * 16, 16)] = zvec

    base_r = s * _RPT
    for t in range(_RPT // _C):
        pltpu.sync_copy(wvb, acc_v.at[pl.ds(base_r + t * _C, _C)])
    pltpu.sync_copy(wvb, acc_d.at[pl.ds(s * _DRPT, _DRPT)])
    plsc.subcore_barrier()

    ew_base = wid * _EPW

    def _chunk(ci, carry):
        base = ew_base + ci * _C
        pltpu.sync_copy(src_hbm.at[pl.ds(base, _C)], src_v)
        pltpu.sync_copy(dst_hbm.at[pl.ds(base, _C)], dst_v)
        pltpu.sync_copy(d16_hbm.at[pl.ds(base, _C)], d16_v)
        pltpu.sync_copy(a0_hbm.at[pl.ds(base, _C)], a0_v)
        pltpu.sync_copy(r8_hbm.at[pl.ds(base, _C)], r8_v)
        cp1 = pltpu.async_copy(k_hbm.at[src_v], kb, sem)
        cp2 = pltpu.async_copy(q_hbm.at[dst_v], qb, sem2)
        cp3 = pltpu.async_copy(v_hbm.at[src_v], vb, sem3)
        cp1.wait()
        cp2.wait()
        cp3.wait()

        cp1 = pltpu.async_copy(k_hbm.at[src_v], kb, sem)
        cp2 = pltpu.async_copy(q_hbm.at[dst_v], qb, sem2)
        cp3 = pltpu.async_copy(v_hbm.at[src_v], vb, sem3)
        cp1.wait()
        cp2.wait()
        cp3.wait()

        def _didx(g, carry):
            dv = dst_v[pl.ds(g * 16, 16)]
            a0_v[pl.ds(g * 16, 16)] = lax.shift_left(
                lax.shift_right_logical(dv & 15, 1), 4)
            r8_v[pl.ds(g * 16, 16)] = lax.shift_left(dv & 1, 3)
            return carry
        lax.fori_loop(0, _C // 16, _didx, 0)


        def _prod(e, carry):
            for h in range(_H):
                prod = kb[e, pl.ds(h * 16, 16)] * qb[e, pl.ds(h * 16, 16)]
                idx = (h * 16 + iota16) * _CS + e
                plsc.store_scatter(pb, [idx], prod)
            return carry
        lax.fori_loop(0, _C, _prod, 0)

        def _dot(t, carry):
            g = t // _H
            h = t % _H
            acc = jnp.zeros((16,), jnp.float32)
            for d in range(16):
                acc = acc + pb[pl.ds((h * 16 + d) * _CS + g * 16, 16)]
            ee = jnp.exp(acc)
            ee_t[pl.ds(h * _CS + g * 16, 16)] = ee
            return carry
        lax.fori_loop(0, (_CS // 16) * _H, _dot, 0)

        def _medge(e, carry):
            a0 = plsc.load_gather(a0_v, [jnp.zeros((16,), jnp.int32) + e])
            r8 = plsc.load_gather(r8_v, [jnp.zeros((16,), jnp.int32) + e])
            valid = (iota16 >= r8) & (iota16 < r8 + 8)
            hidx = jnp.where(valid, iota16 - r8, _H)
            w = plsc.load_gather(ee_t, [hidx * _CS + e])
            for j in range(_D // 16):
                db[e, pl.ds(j * 16, 16)] = jnp.where(a0 == j * 16, w, zvec)
            for h in range(_H):
                eev = plsc.load_gather(
                    ee_t, [jnp.full((16,), h * _CS, jnp.int32) + e])
                wvb[e, pl.ds(h * 16, 16)] = vb[e, pl.ds(h * 16, 16)] * eev
            return carry
        lax.fori_loop(0, _C, _medge, 0)

        pltpu.sync_copy(wvb, acc_v.at[dst_v], add=True)
        pltpu.sync_copy(db, acc_d.at[d16_v], add=True)
        return carry
    lax.fori_loop(0, _NCH, _chunk, 0)

    plsc.subcore_barrier()
    for t in range(_RPT // _C):
        r0 = base_r + t * _C
        pltpu.sync_copy(acc_v.at[pl.ds(r0, _C)], wvb)
        pltpu.sync_copy(wvb, out_v_hbm.at[c, pl.ds(r0, _C)])
    d0 = s * _DRPT
    pltpu.sync_copy(acc_d.at[pl.ds(d0, _DRPT)], wvb)
    pltpu.sync_copy(wvb, out_d_hbm.at[c, pl.ds(d0, _DRPT)])


def _edge(q, k, v, src, dst, d16, a0, r8):
    mesh = plsc.VectorSubcoreMesh(core_axis_name="c", subcore_axis_name="s")
    fn = pl.kernel(
        _edge_body,
        out_type=[
            jax.ShapeDtypeStruct((2, _NPAD, _D), jnp.float32),
            jax.ShapeDtypeStruct((2, _DR, _D), jnp.float32),
        ],
        mesh=mesh,
        compiler_params=pltpu.CompilerParams(needs_layout_passes=False),
        scratch_types=[
            pltpu.VMEM((_C,), jnp.int32),
            pltpu.VMEM((_C,), jnp.int32),
            pltpu.VMEM((_C,), jnp.int32),
            pltpu.VMEM((_C,), jnp.int32),
            pltpu.VMEM((_C,), jnp.int32),
            pltpu.VMEM((_C, _D), jnp.float32),
            pltpu.VMEM((_C, _D), jnp.float32),
            pltpu.VMEM((_C, _D), jnp.float32),
            pltpu.VMEM((_CS * _D,), jnp.float32),
            pltpu.VMEM(((_H + 1) * _CS,), jnp.float32),
            pltpu.VMEM((_C, _D), jnp.float32),
            pltpu.VMEM((_C, _D), jnp.float32),
            pltpu.VMEM_SHARED((_NPAD, _D), jnp.float32),
            pltpu.VMEM_SHARED((_DR, _D), jnp.float32),
            pltpu.SemaphoreType.DMA,
            pltpu.SemaphoreType.DMA,
            pltpu.SemaphoreType.DMA,
        ],
    )
    return fn(q, k, v, src, dst, d16, a0, r8)


def _ln(x, g, b):
    mu = jnp.mean(x, axis=1, keepdims=True)
    var = jnp.mean((x - mu) * (x - mu), axis=1, keepdims=True)
    return (x - mu) * lax.rsqrt(var + 1e-5) * g + b


def _comb_body(pv_ref, pd_ref, feat_ref, lng_ref, lnb_ref, w1_ref, b1_ref,
               al_ref, w2_ref, b2_ref, o_ref):
    pv = pv_ref[...]
    sacc = pv[0] + pv[1]
    pd = pd_ref[...]
    den = pd[0] + pd[1]
    den = jnp.where(den == 0.0, jnp.float32(1.0), den)
    hh = lax.broadcasted_iota(jnp.int32, (_H, _D), 0)
    jj = lax.broadcasted_iota(jnp.int32, (_H, _D), 1)
    expand = (jj // _DH == hh).astype(jnp.float32)
    den16 = jnp.dot(den, expand, preferred_element_type=jnp.float32)
    rst = sacc / den16 + feat_ref[...]
    g = lng_ref[...]
    b = lnb_ref[...]
    rst = _ln(rst, g, b)
    h1 = jnp.dot(rst, w1_ref[...], preferred_element_type=jnp.float32) + b1_ref[...]
    h1 = jnp.where(h1 >= 0, h1, al_ref[...] * h1)
    ffn = jnp.dot(h1, w2_ref[...], preferred_element_type=jnp.float32) + b2_ref[...]
    o_ref[...] = _ln(rst + ffn, g, b)


def _combine(pv, pd, feat, ln_g, ln_b, W1, bf1, alpha, W2, bf2):
    grid = (_N // 1000,)
    return pl.pallas_call(
        _comb_body,
        grid=grid,
        in_specs=[
            pl.BlockSpec((2, 1000, _D), lambda i: (0, i, 0)),
            pl.BlockSpec((2, 1000, _H), lambda i: (0, i, 0)),
            pl.BlockSpec((1000, _D), lambda i: (i, 0)),
            pl.BlockSpec((1, _D), lambda i: (0, 0)),
            pl.BlockSpec((1, _D), lambda i: (0, 0)),
            pl.BlockSpec((_D, _DFF), lambda i: (0, 0)),
            pl.BlockSpec((1, _DFF), lambda i: (0, 0)),
            pl.BlockSpec((1, _DFF), lambda i: (0, 0)),
            pl.BlockSpec((_DFF, _D), lambda i: (0, 0)),
            pl.BlockSpec((1, _D), lambda i: (0, 0)),
        ],
        out_specs=pl.BlockSpec((1000, _D), lambda i: (i, 0)),
        out_shape=jax.ShapeDtypeStruct((_N, _D), jnp.float32),
    )(pv, pd, feat, ln_g, ln_b, W1, bf1, alpha, W2, bf2)


def kernel(feat, edge_index, Wq, bq, Wk, bk, Wv, bv, ln_g, ln_b, W1, bf1,
           alpha, W2, bf2):
    ei = edge_index.astype(jnp.int32)
    dst3 = ei[1].reshape(10, 1, _E // 10)
    q, k, v, d16_3d, a0_3d, r8_3d = _proj(
        feat, Wq, bq.reshape(1, _D), Wk, bk.reshape(1, _D),
        Wv, bv.reshape(1, _D), dst3)
    src = ei[0]
    dst = ei[1]
    pv, pd_raw = _edge(q, k, v, src, dst, d16_3d.reshape(_E),
                       a0_3d.reshape(_E), r8_3d.reshape(_E))
    pd = pd_raw.reshape(2, _NPAD, _H)
    return _combine(pv, pd, feat, ln_g.reshape(1, _D), ln_b.reshape(1, _D),
                    W1, bf1.reshape(1, _DFF), alpha.reshape(1, _DFF),
                    W2, bf2.reshape(1, _D))
